# Initial kernel scaffold; baseline (speedup 1.0000x reference)
import jax
import jax.numpy as jnp
from jax.experimental import pallas as pl


def kernel(contour, cnn_feature):
    raise NotImplementedError("write your pallas kernel here")


# speedup vs baseline: 9.8827x; 9.8827x over previous
import functools

import jax
import jax.numpy as jnp
from jax import lax
from jax.experimental import pallas as pl
from jax.experimental.pallas import tpu as pltpu
from jax.experimental.pallas import tpu_sc as plsc

_P = 16
_E = 128
_H = 256
_W = 256
_LANES = 16
_NTEC = 32
_TASKS = _P * (_H // _LANES)
_TASK_WORDS = _LANES * _W


def _sc_hist_body(edges_hbm, hist_hbm, edges_v, hist_v):
  wid = lax.axis_index("s") * 2 + lax.axis_index("c")
  pltpu.sync_copy(edges_hbm, edges_v)
  lane = lax.iota(jnp.int32, _LANES)
  ones = jnp.ones((_LANES,), jnp.float32)
  zeros = jnp.zeros((_LANES,), jnp.float32)

  for k in range(_TASKS // _NTEC):
    task = wid * (_TASKS // _NTEC) + k
    p = task // (_H // _LANES)
    rg = task % (_H // _LANES)
    py = (lane + rg * _LANES).astype(jnp.float32)
    row_base = lane * _W

    def zbody(i, _):
      hist_v[pl.ds(pl.multiple_of(i * _LANES, _LANES), _LANES)] = zeros
      return 0
    lax.fori_loop(0, _TASK_WORDS // _LANES, zbody, 0)

    def ebody(e, _):
      base = jnp.full((_LANES,), p * (4 * _E) + e, jnp.int32)
      ax = plsc.load_gather(edges_v, [base])
      ay = plsc.load_gather(edges_v, [base + _E])
      bx = plsc.load_gather(edges_v, [base + 2 * _E])
      by = plsc.load_gather(edges_v, [base + 3 * _E])
      cond = (ay > py) != (by > py)
      t = (py - ay) / (by - ay + 1e-9)
      xint = ax + t * (bx - ax)
      it = xint.astype(jnp.int32)
      c = it + (xint > it.astype(jnp.float32)).astype(jnp.int32)
      valid = cond & (c < _W)
      cc = jnp.minimum(jnp.maximum(c, 0), _W - 1)
      plsc.addupdate_scatter(hist_v, [row_base + cc], ones, mask=valid)
      return 0
    lax.fori_loop(0, _E, ebody, 0)

    pltpu.sync_copy(hist_v, hist_hbm.at[pl.ds(task * _TASK_WORDS, _TASK_WORDS)])


_sc_hist = functools.partial(
    pl.kernel,
    out_type=jax.ShapeDtypeStruct((_TASKS * _TASK_WORDS,), jnp.float32),
    mesh=plsc.VectorSubcoreMesh(core_axis_name="c", subcore_axis_name="s"),
    compiler_params=pltpu.CompilerParams(needs_layout_passes=False),
    scratch_types=[
        pltpu.VMEM((_P * 4 * _E,), jnp.float32),
        pltpu.VMEM((_TASK_WORDS,), jnp.float32),
    ],
)(_sc_hist_body)


_YB = 32


def _tc_fuse_body(hist_ref, feat_ref, mask_ref, out_ref):
  hist = hist_ref[...]
  bi = lax.broadcasted_iota(jnp.int32, (_W, _W), 0)
  xi = lax.broadcasted_iota(jnp.int32, (_W, _W), 1)
  tri = (bi <= xi).astype(jnp.float32)
  cnt = jnp.dot(hist.reshape(_P * _YB, _W), tri,
                preferred_element_type=jnp.float32)
  par = cnt - 2.0 * jnp.floor(cnt * 0.5)
  mask = par.reshape(_P, _YB, _W)
  mask_ref[...] = mask
  mm = jnp.max(mask, axis=0)
  f = feat_ref[...]
  out_ref[...] = jnp.maximum(mm[None] * f + f, 0.0)


def kernel(contour, cnn_feature):
  bs, c_in, h, w = cnn_feature.shape
  x1 = jnp.clip(contour[..., 0], 0.0, float(w - 1))
  y1 = jnp.clip(contour[..., 1], 0.0, float(h - 1))
  x2 = jnp.roll(x1, -1, axis=1)
  y2 = jnp.roll(y1, -1, axis=1)
  edges = jnp.stack([x1, y1, x2, y2], axis=1).reshape(-1)

  hist = _sc_hist(edges).reshape(_P, _H, _W)

  mask, fused = pl.pallas_call(
      _tc_fuse_body,
      grid=(_H // _YB,),
      in_specs=[
          pl.BlockSpec((_P, _YB, _W), lambda i: (0, i, 0)),
          pl.BlockSpec((c_in, _YB, _W), lambda i: (0, i, 0)),
      ],
      out_specs=[
          pl.BlockSpec((_P, _YB, _W), lambda i: (0, i, 0)),
          pl.BlockSpec((c_in, _YB, _W), lambda i: (0, i, 0)),
      ],
      out_shape=[
          jax.ShapeDtypeStruct((_P, _H, _W), jnp.float32),
          jax.ShapeDtypeStruct((c_in, _H, _W), jnp.float32),
      ],
  )(hist, cnn_feature[0])

  return mask[None], fused[None]


# speedup vs baseline: 12.4761x; 1.2624x over previous
import functools

import jax
import jax.numpy as jnp
from jax import lax
from jax.experimental import pallas as pl
from jax.experimental.pallas import tpu as pltpu
from jax.experimental.pallas import tpu_sc as plsc

_P = 16
_E = 128
_H = 256
_W = 256
_LANES = 16
_NTEC = 32
_TASKS = _P * (_H // _LANES)
_TASK_WORDS = _LANES * _W


def _sc_hist_body(edges_hbm, hist_hbm, edges_v, hist_v):
  wid = lax.axis_index("s") * 2 + lax.axis_index("c")
  pltpu.sync_copy(edges_hbm, edges_v)
  lane = lax.iota(jnp.int32, _LANES)
  ones = jnp.ones((_LANES,), jnp.float32)
  zeros = jnp.zeros((_LANES,), jnp.float32)

  for k in range(_TASKS // _NTEC):
    task = wid * (_TASKS // _NTEC) + k
    p = task // (_H // _LANES)
    rg = task % (_H // _LANES)
    py = (lane + rg * _LANES).astype(jnp.float32)

    def zbody(i, _):
      col = pl.multiple_of(i * _LANES, _LANES)
      for j in range(_LANES):
        hist_v[j, pl.ds(col, _LANES)] = zeros
      return 0
    lax.fori_loop(0, _W // _LANES, zbody, 0)

    def ebody(e, _):
      base = jnp.full((_LANES,), p * (5 * _E) + e, jnp.int32)
      ay = plsc.load_gather(edges_v, [base])
      by = plsc.load_gather(edges_v, [base + _E])
      ax = plsc.load_gather(edges_v, [base + 2 * _E])
      dx = plsc.load_gather(edges_v, [base + 3 * _E])
      inv = plsc.load_gather(edges_v, [base + 4 * _E])
      cond = (ay > py) != (by > py)
      xint = ax + ((py - ay) * inv) * dx
      it = xint.astype(jnp.int32)
      c = it + (xint > it.astype(jnp.float32)).astype(jnp.int32)
      valid = cond & (c < _W)
      cc = jnp.minimum(jnp.maximum(c, 0), _W - 1)
      plsc.addupdate_scatter(hist_v, [lane, cc], ones, mask=valid)
      return 0
    lax.fori_loop(0, _E, ebody, 0, unroll=4)

    pltpu.sync_copy(hist_v, hist_hbm.at[p, pl.ds(rg * _LANES, _LANES), :])


_sc_hist = functools.partial(
    pl.kernel,
    out_type=jax.ShapeDtypeStruct((_P, _H, _W), jnp.float32),
    mesh=plsc.VectorSubcoreMesh(core_axis_name="c", subcore_axis_name="s"),
    compiler_params=pltpu.CompilerParams(needs_layout_passes=False),
    scratch_types=[
        pltpu.VMEM((_P * 5 * _E,), jnp.float32),
        pltpu.VMEM((_LANES, _W), jnp.float32),
    ],
)(_sc_hist_body)


_YB = 32


def _tc_fuse_body(hist_ref, feat_ref, mask_ref, out_ref):
  hist = hist_ref[...]
  bi = lax.broadcasted_iota(jnp.int32, (_W, _W), 0)
  xi = lax.broadcasted_iota(jnp.int32, (_W, _W), 1)
  tri = (bi <= xi).astype(jnp.float32)
  cnt = jnp.dot(hist.reshape(_P * _YB, _W), tri,
                preferred_element_type=jnp.float32)
  par = cnt - 2.0 * jnp.floor(cnt * 0.5)
  mask = par.reshape(_P, _YB, _W)
  mask_ref[...] = mask
  mm = jnp.max(mask, axis=0)
  f = feat_ref[...]
  out_ref[...] = jnp.maximum(mm[None] * f + f, 0.0)


def kernel(contour, cnn_feature):
  bs, c_in, h, w = cnn_feature.shape
  x1 = jnp.clip(contour[..., 0], 0.0, float(w - 1))
  y1 = jnp.clip(contour[..., 1], 0.0, float(h - 1))
  x2 = jnp.roll(x1, -1, axis=1)
  y2 = jnp.roll(y1, -1, axis=1)
  inv = 1.0 / (y2 - y1 + 1e-9)
  edges = jnp.stack([y1, y2, x1, x2 - x1, inv], axis=1).reshape(-1)

  hist = _sc_hist(edges)

  mask, fused = pl.pallas_call(
      _tc_fuse_body,
      grid=(_H // _YB,),
      in_specs=[
          pl.BlockSpec((_P, _YB, _W), lambda i: (0, i, 0)),
          pl.BlockSpec((c_in, _YB, _W), lambda i: (0, i, 0)),
      ],
      out_specs=[
          pl.BlockSpec((_P, _YB, _W), lambda i: (0, i, 0)),
          pl.BlockSpec((c_in, _YB, _W), lambda i: (0, i, 0)),
      ],
      out_shape=[
          jax.ShapeDtypeStruct((_P, _H, _W), jnp.float32),
          jax.ShapeDtypeStruct((c_in, _H, _W), jnp.float32),
      ],
  )(hist, cnn_feature[0])

  return mask[None], fused[None]


# speedup vs baseline: 16.2804x; 1.3049x over previous
import functools

import jax
import jax.numpy as jnp
from jax import lax
from jax.experimental import pallas as pl
from jax.experimental.pallas import tpu as pltpu
from jax.experimental.pallas import tpu_sc as plsc

_P = 16
_E = 128
_H = 256
_W = 256
_LANES = 16
_HHALF = _H // 2
_NCH = _HHALF // _LANES


def _ceil_i32(x):
  it = x.astype(jnp.int32)
  return it + (x > it.astype(jnp.float32)).astype(jnp.int32)


def _sc_hist_body(cont_hbm, hist_hbm, cont_v, comp_v, bnd_v, hist_v):
  wid = lax.axis_index("s") * 2 + lax.axis_index("c")
  p = wid // 2
  half = wid % 2
  ylo_half = half * _HHALF

  pltpu.sync_copy(cont_hbm.at[pl.ds(p * (2 * _E), 2 * _E)], cont_v)

  lane = lax.iota(jnp.int32, _LANES)
  ones = jnp.ones((_LANES,), jnp.float32)
  zeros = jnp.zeros((_LANES,), jnp.float32)
  fmax = float(_W - 1)

  for j in range(_E // _LANES):
    i0 = 2 * _LANES * j + 2 * lane
    i1 = (i0 + 2) & (2 * _E - 1)
    gx = plsc.load_gather(cont_v, [i0])
    gy = plsc.load_gather(cont_v, [i0 + 1])
    hx = plsc.load_gather(cont_v, [i1])
    hy = plsc.load_gather(cont_v, [(i0 + 3) & (2 * _E - 1)])
    ax = jnp.minimum(jnp.maximum(gx, 0.0), fmax)
    ay = jnp.minimum(jnp.maximum(gy, 0.0), fmax)
    bx = jnp.minimum(jnp.maximum(hx, 0.0), fmax)
    by = jnp.minimum(jnp.maximum(hy, 0.0), fmax)
    comp_v[pl.ds(0 * _E + j * _LANES, _LANES)] = ay
    comp_v[pl.ds(1 * _E + j * _LANES, _LANES)] = by
    comp_v[pl.ds(2 * _E + j * _LANES, _LANES)] = ax
    comp_v[pl.ds(3 * _E + j * _LANES, _LANES)] = bx - ax
    comp_v[pl.ds(4 * _E + j * _LANES, _LANES)] = 1.0 / (by - ay + 1e-9)
    ymin = jnp.minimum(ay, by)
    ymax = jnp.maximum(ay, by)
    ylo = jnp.maximum(_ceil_i32(ymin), ylo_half)
    yhi = jnp.minimum(_ceil_i32(ymax) - 1, ylo_half + _HHALF - 1)
    kf = lax.shift_right_arithmetic(ylo, 4)
    kl = lax.shift_right_arithmetic(yhi, 4)
    cnt = jnp.where(yhi >= ylo, jnp.maximum(kl - kf + 1, 0), 0)
    kfloc = kf - half * _NCH
    bnd_v[pl.ds(j * _LANES, _LANES)] = kfloc * _LANES + cnt

  def zbody(r, _):
    for cb in range(_W // _LANES):
      hist_v[r, pl.ds(cb * _LANES, _LANES)] = zeros
    return 0
  lax.fori_loop(0, _HHALF, zbody, 0)

  def ebody(e, _):
    es = jnp.full((_LANES,), e, jnp.int32)
    pk = jnp.max(plsc.load_gather(bnd_v, [es]))
    cnt = jnp.bitwise_and(pk, _LANES - 1)
    kf = lax.shift_right_arithmetic(pk, 4)

    @pl.when(cnt > 0)
    def _():
      ay = plsc.load_gather(comp_v, [es])
      by = plsc.load_gather(comp_v, [es + _E])
      ax = plsc.load_gather(comp_v, [es + 2 * _E])
      dx = plsc.load_gather(comp_v, [es + 3 * _E])
      inv = plsc.load_gather(comp_v, [es + 4 * _E])

      def cbody(i, _):
        rowloc = (kf + i) * _LANES + lane
        py = (rowloc + ylo_half).astype(jnp.float32)
        cond = (ay > py) != (by > py)
        xint = ax + ((py - ay) * inv) * dx
        c = _ceil_i32(xint)
        valid = cond & (c < _W)
        cc = jnp.minimum(jnp.maximum(c, 0), _W - 1)
        plsc.addupdate_scatter(hist_v, [rowloc, cc], ones, mask=valid)
        return 0
      lax.fori_loop(0, cnt, cbody, 0)
    return 0
  lax.fori_loop(0, _E, ebody, 0)

  pltpu.sync_copy(hist_v, hist_hbm.at[p, pl.ds(half * _HHALF, _HHALF), :])


_sc_hist = functools.partial(
    pl.kernel,
    out_type=jax.ShapeDtypeStruct((_P, _H, _W), jnp.float32),
    mesh=plsc.VectorSubcoreMesh(core_axis_name="c", subcore_axis_name="s"),
    compiler_params=pltpu.CompilerParams(needs_layout_passes=False),
    scratch_types=[
        pltpu.VMEM((2 * _E,), jnp.float32),
        pltpu.VMEM((5 * _E,), jnp.float32),
        pltpu.VMEM((_E,), jnp.int32),
        pltpu.VMEM((_HHALF, _W), jnp.float32),
    ],
)(_sc_hist_body)


_YB = 32


def _tc_fuse_body(hist_ref, feat_ref, mask_ref, out_ref):
  hist = hist_ref[...]
  bi = lax.broadcasted_iota(jnp.int32, (_W, _W), 0)
  xi = lax.broadcasted_iota(jnp.int32, (_W, _W), 1)
  tri = (bi <= xi).astype(jnp.float32)
  cnt = jnp.dot(hist.reshape(_P * _YB, _W), tri,
                preferred_element_type=jnp.float32)
  par = cnt - 2.0 * jnp.floor(cnt * 0.5)
  mask = par.reshape(_P, _YB, _W)
  mask_ref[...] = mask
  mm = jnp.max(mask, axis=0)
  f = feat_ref[...]
  out_ref[...] = jnp.maximum(mm[None] * f + f, 0.0)


def kernel(contour, cnn_feature):
  bs, c_in, h, w = cnn_feature.shape
  hist = _sc_hist(contour.reshape(-1))

  mask, fused = pl.pallas_call(
      _tc_fuse_body,
      grid=(_H // _YB,),
      in_specs=[
          pl.BlockSpec((_P, _YB, _W), lambda i: (0, i, 0)),
          pl.BlockSpec((c_in, _YB, _W), lambda i: (0, i, 0)),
      ],
      out_specs=[
          pl.BlockSpec((_P, _YB, _W), lambda i: (0, i, 0)),
          pl.BlockSpec((c_in, _YB, _W), lambda i: (0, i, 0)),
      ],
      out_shape=[
          jax.ShapeDtypeStruct((_P, _H, _W), jnp.float32),
          jax.ShapeDtypeStruct((c_in, _H, _W), jnp.float32),
      ],
  )(hist, cnn_feature[0])

  return mask[None], fused[None]


# speedup vs baseline: 16.6871x; 1.0250x over previous
import functools

import jax
import jax.numpy as jnp
from jax import lax
from jax.experimental import pallas as pl
from jax.experimental.pallas import tpu as pltpu
from jax.experimental.pallas import tpu_sc as plsc

_P = 16
_E = 128
_H = 256
_W = 256
_LANES = 16
_HHALF = _H // 2
_NCH = _HHALF // _LANES


def _ceil_i32(x):
  it = x.astype(jnp.int32)
  return it + (x > it.astype(jnp.float32)).astype(jnp.int32)


def _sc_hist_body(cont_hbm, hist_hbm, cont_v, comp_v, bnd_v, hist_v):
  wid = lax.axis_index("s") * 2 + lax.axis_index("c")
  p = wid // 2
  half = wid % 2
  ylo_half = half * _HHALF

  pltpu.sync_copy(cont_hbm.at[pl.ds(p * (2 * _E), 2 * _E)], cont_v)

  lane = lax.iota(jnp.int32, _LANES)
  ones = jnp.ones((_LANES,), jnp.float32)
  zeros = jnp.zeros((_LANES,), jnp.float32)
  zeros_i = jnp.zeros((_LANES,), jnp.int32)
  ones_i = jnp.ones((_LANES,), jnp.int32)
  fmax = float(_W - 1)

  bnd_v[pl.ds(_E, _LANES)] = zeros_i

  for j in range(_E // _LANES):
    i0 = 2 * _LANES * j + 2 * lane
    i1 = (i0 + 2) & (2 * _E - 1)
    gx = plsc.load_gather(cont_v, [i0])
    gy = plsc.load_gather(cont_v, [i0 + 1])
    hx = plsc.load_gather(cont_v, [i1])
    hy = plsc.load_gather(cont_v, [(i0 + 3) & (2 * _E - 1)])
    ax = jnp.minimum(jnp.maximum(gx, 0.0), fmax)
    ay = jnp.minimum(jnp.maximum(gy, 0.0), fmax)
    bx = jnp.minimum(jnp.maximum(hx, 0.0), fmax)
    by = jnp.minimum(jnp.maximum(hy, 0.0), fmax)
    comp_v[pl.ds(0 * _E + j * _LANES, _LANES)] = ay
    comp_v[pl.ds(1 * _E + j * _LANES, _LANES)] = by
    comp_v[pl.ds(2 * _E + j * _LANES, _LANES)] = ax
    comp_v[pl.ds(3 * _E + j * _LANES, _LANES)] = bx - ax
    comp_v[pl.ds(4 * _E + j * _LANES, _LANES)] = 1.0 / (by - ay + 1e-9)
    ymin = jnp.minimum(ay, by)
    ymax = jnp.maximum(ay, by)
    ylo = jnp.maximum(_ceil_i32(ymin), ylo_half)
    yhi = jnp.minimum(_ceil_i32(ymax) - 1, ylo_half + _HHALF - 1)
    kf = lax.shift_right_arithmetic(ylo, 4)
    kl = lax.shift_right_arithmetic(yhi, 4)
    cnt = jnp.where(yhi >= ylo, jnp.maximum(kl - kf + 1, 0), 0)
    kfloc = kf - half * _NCH
    bnd_v[pl.ds(j * _LANES, _LANES)] = kfloc * _LANES + cnt

  def zbody(r, _):
    for cb in range(_W // _LANES):
      hist_v[r, pl.ds(cb * _LANES, _LANES)] = zeros
    return 0
  lax.fori_loop(0, _HHALF, zbody, 0)

  def ebody(e, _):
    es = jnp.full((_LANES,), e, jnp.int32)
    pk = jnp.max(plsc.load_gather(bnd_v, [es]))
    cnt = jnp.bitwise_and(pk, _LANES - 1)
    kf = lax.shift_right_arithmetic(pk, 4)

    @pl.when(cnt > 0)
    def _():
      ay = plsc.load_gather(comp_v, [es])
      by = plsc.load_gather(comp_v, [es + _E])
      ax = plsc.load_gather(comp_v, [es + 2 * _E])
      dx = plsc.load_gather(comp_v, [es + 3 * _E])
      inv = plsc.load_gather(comp_v, [es + 4 * _E])

      def cbody(i, _):
        rowloc = (kf + i) * _LANES + lane
        py = (rowloc + ylo_half).astype(jnp.float32)
        cond = (ay > py) != (by > py)
        xint = ax + ((py - ay) * inv) * dx
        c = _ceil_i32(xint)
        valid = cond & (c < _W)
        cc = jnp.minimum(jnp.maximum(c, 0), _W - 1)
        plsc.addupdate_scatter(hist_v, [rowloc, cc], ones, mask=valid)
        return 0
      lax.fori_loop(0, cnt, cbody, 0)
    return 0
  lax.fori_loop(0, _E, ebody, 0)

  pltpu.sync_copy(hist_v, hist_hbm.at[p, pl.ds(half * _HHALF, _HHALF), :])


_sc_hist = functools.partial(
    pl.kernel,
    out_type=jax.ShapeDtypeStruct((_P, _H, _W), jnp.float32),
    mesh=plsc.VectorSubcoreMesh(core_axis_name="c", subcore_axis_name="s"),
    compiler_params=pltpu.CompilerParams(needs_layout_passes=False),
    scratch_types=[
        pltpu.VMEM((2 * _E,), jnp.float32),
        pltpu.VMEM((5 * _E,), jnp.float32),
        pltpu.VMEM((_E + _LANES,), jnp.int32),
        pltpu.VMEM((_HHALF, _W), jnp.float32),
    ],
)(_sc_hist_body)


_YB = 64


def _tc_fuse_body(hist_ref, feat_ref, mask_ref, out_ref):
  hist = hist_ref[...]
  bi = lax.broadcasted_iota(jnp.int32, (_W, _W), 0)
  xi = lax.broadcasted_iota(jnp.int32, (_W, _W), 1)
  tri = (bi <= xi).astype(jnp.float32)
  cnt = jnp.dot(hist.reshape(_P * _YB, _W), tri,
                preferred_element_type=jnp.float32)
  par = cnt - 2.0 * jnp.floor(cnt * 0.5)
  mask = par.reshape(_P, _YB, _W)
  mask_ref[...] = mask
  mm = jnp.max(mask, axis=0)
  f = feat_ref[...]
  out_ref[...] = jnp.maximum(mm[None] * f + f, 0.0)


def kernel(contour, cnn_feature):
  bs, c_in, h, w = cnn_feature.shape
  hist = _sc_hist(contour.reshape(-1))

  mask, fused = pl.pallas_call(
      _tc_fuse_body,
      grid=(_H // _YB,),
      in_specs=[
          pl.BlockSpec((_P, _YB, _W), lambda i: (0, i, 0)),
          pl.BlockSpec((c_in, _YB, _W), lambda i: (0, i, 0)),
      ],
      out_specs=[
          pl.BlockSpec((_P, _YB, _W), lambda i: (0, i, 0)),
          pl.BlockSpec((c_in, _YB, _W), lambda i: (0, i, 0)),
      ],
      out_shape=[
          jax.ShapeDtypeStruct((_P, _H, _W), jnp.float32),
          jax.ShapeDtypeStruct((c_in, _H, _W), jnp.float32),
      ],
  )(hist, cnn_feature[0])

  return mask[None], fused[None]
